# baseline (speedup 1.0000x reference)
import jax
import jax.numpy as jnp
from jax.experimental import pallas as pl
from jax.experimental.pallas import tpu as pltpu

_D_MODEL = 4096
_N_EXPERTS = 64
_TOP_K = 8
_CH = 256
_NBUF = 6
_STRIP = 64


def _chunk_compute(x, wt, iota, probs_ref, idx_ref, w_ref, base):
    logits = jnp.dot(x, wt, preferred_element_type=jnp.float32)
    for t in range(_CH // _STRIP):
        strip = logits[t * _STRIP:(t + 1) * _STRIP, :]
        cur = strip
        vals = []
        idxs = []
        for _ in range(_TOP_K):
            mv = jnp.max(cur, axis=-1, keepdims=True)
            ik = jnp.min(jnp.where(cur == mv, iota, float(_N_EXPERTS)),
                         axis=-1, keepdims=True)
            vals.append(mv)
            idxs.append(ik)
            cur = jnp.where(iota == ik, -jnp.inf, cur)
        m = vals[0]
        e = jnp.exp(strip - m)
        s = jnp.sum(e, axis=-1, keepdims=True)
        probs = e / s
        rows = pl.ds(base + t * _STRIP, _STRIP)
        probs_ref[rows, :] = probs
        top_lv = jnp.concatenate(vals, axis=-1)
        top_idx = jnp.concatenate(idxs, axis=-1).astype(jnp.int32)
        top_vals = jnp.exp(top_lv - m) / s
        top_vals = top_vals / (jnp.sum(top_vals, axis=-1, keepdims=True)
                               + 1e-9)
        idx_ref[rows, :] = top_idx
        w_ref[rows, :] = top_vals


def _router_body(x_hbm, wt_ref, probs_ref, idx_ref, w_ref, xbuf, sems):
    n_chunks = x_hbm.shape[0] // _CH
    wt = wt_ref[...]
    iota = jax.lax.broadcasted_iota(
        jnp.int32, (_STRIP, _N_EXPERTS), 1).astype(jnp.float32)

    def _start(c, slot):
        pltpu.make_async_copy(
            x_hbm.at[pl.ds(c * _CH, _CH), :],
            xbuf.at[slot],
            sems.at[slot],
        ).start()

    def _wait(c, slot):
        pltpu.make_async_copy(
            x_hbm.at[pl.ds(c * _CH, _CH), :],
            xbuf.at[slot],
            sems.at[slot],
        ).wait()

    for c in range(_NBUF):
        _start(c, c)

    def _step(c, carry):
        slot = jax.lax.rem(c, _NBUF)
        _wait(c, slot)
        x = xbuf[slot]
        _chunk_compute(x, wt, iota, probs_ref, idx_ref, w_ref, c * _CH)

        @pl.when(c + _NBUF < n_chunks)
        def _():
            _start(c + _NBUF, slot)

        return carry

    jax.lax.fori_loop(0, n_chunks, _step, 0)


def kernel(x, W):
    n_tokens = x.shape[0]
    wt = W.T
    out_shapes = (
        jax.ShapeDtypeStruct((n_tokens, _N_EXPERTS), jnp.float32),
        jax.ShapeDtypeStruct((n_tokens, _TOP_K), jnp.int32),
        jax.ShapeDtypeStruct((n_tokens, _TOP_K), jnp.float32),
    )
    probs, idx, w = pl.pallas_call(
        _router_body,
        in_specs=[
            pl.BlockSpec(memory_space=pl.ANY),
            pl.BlockSpec(memory_space=pltpu.MemorySpace.VMEM),
        ],
        out_specs=(
            pl.BlockSpec(memory_space=pltpu.MemorySpace.VMEM),
            pl.BlockSpec(memory_space=pltpu.MemorySpace.VMEM),
            pl.BlockSpec(memory_space=pltpu.MemorySpace.VMEM),
        ),
        out_shape=out_shapes,
        scratch_shapes=[
            pltpu.VMEM((_NBUF, _CH, _D_MODEL), jnp.float32),
            pltpu.SemaphoreType.DMA((_NBUF,)),
        ],
    )(x, wt)
    return (idx, w, probs)


# speedup vs baseline: 1.3955x; 1.3955x over previous
import jax
import jax.numpy as jnp
from jax.experimental import pallas as pl
from jax.experimental.pallas import tpu as pltpu

_D_MODEL = 4096
_N_EXPERTS = 64
_TOP_K = 8
_BT = 1024
_N_SPLIT = 4
_BS = _BT // _N_SPLIT
_STRIP = 64


def _router_body(*refs):
    x_refs = refs[:_N_SPLIT]
    wt_ref, probs_ref, idx_ref, w_ref = refs[_N_SPLIT:]
    wt = wt_ref[...]
    iota = jax.lax.broadcasted_iota(
        jnp.int32, (_STRIP, _N_EXPERTS), 1).astype(jnp.float32)
    for c in range(_N_SPLIT):
        x = x_refs[c][...]
        logits = jnp.dot(x, wt, preferred_element_type=jnp.float32)
        for t in range(_BS // _STRIP):
            strip = logits[t * _STRIP:(t + 1) * _STRIP, :]
            cur = strip
            vals = []
            idxs = []
            for k in range(_TOP_K):
                mv = jnp.max(cur, axis=-1, keepdims=True)
                ik = jnp.min(jnp.where(cur == mv, iota, float(_N_EXPERTS)),
                             axis=-1, keepdims=True)
                vals.append(mv)
                idxs.append(ik)
                if k + 1 < _TOP_K:
                    cur = jnp.where(iota == ik, -jnp.inf, cur)
            m = vals[0]
            e = jnp.exp(strip - m)
            s = jnp.sum(e, axis=-1, keepdims=True)
            probs = e / s
            rows = pl.ds(c * _BS + t * _STRIP, _STRIP)
            probs_ref[rows, :] = probs
            top_lv = jnp.concatenate(vals, axis=-1)
            top_idx = jnp.concatenate(idxs, axis=-1).astype(jnp.int32)
            top_vals = jnp.exp(top_lv - m) / s
            top_vals = top_vals / (jnp.sum(top_vals, axis=-1, keepdims=True)
                                   + 1e-9)
            idx_ref[rows, :] = top_idx
            w_ref[rows, :] = top_vals


def kernel(x, W):
    n_tokens = x.shape[0]
    grid = (n_tokens // _BT,)
    wt = W.T
    out_shapes = (
        jax.ShapeDtypeStruct((n_tokens, _N_EXPERTS), jnp.float32),
        jax.ShapeDtypeStruct((n_tokens, _TOP_K), jnp.int32),
        jax.ShapeDtypeStruct((n_tokens, _TOP_K), jnp.float32),
    )
    probs, idx, w = pl.pallas_call(
        _router_body,
        grid=grid,
        in_specs=[
            pl.BlockSpec((_BS, _D_MODEL),
                         lambda i, c=c: (_N_SPLIT * i + c, 0))
            for c in range(_N_SPLIT)
        ] + [
            pl.BlockSpec((_D_MODEL, _N_EXPERTS), lambda i: (0, 0)),
        ],
        out_specs=(
            pl.BlockSpec((_BT, _N_EXPERTS), lambda i: (i, 0)),
            pl.BlockSpec((_BT, _TOP_K), lambda i: (i, 0)),
            pl.BlockSpec((_BT, _TOP_K), lambda i: (i, 0)),
        ),
        out_shape=out_shapes,
        compiler_params=pltpu.CompilerParams(
            dimension_semantics=("arbitrary",),
        ),
    )(*([x] * _N_SPLIT), wt)
    return (idx, w, probs)


# speedup vs baseline: 1.4142x; 1.0134x over previous
import jax
import jax.numpy as jnp
from jax.experimental import pallas as pl
from jax.experimental.pallas import tpu as pltpu

_D_MODEL = 4096
_N_EXPERTS = 64
_TOP_K = 8
_BT = 1024
_N_SPLIT = 4
_BS = _BT // _N_SPLIT
_STRIP = 128


def _router_body(*refs):
    x_refs = refs[:_N_SPLIT]
    wt_ref, probs_ref, idx_ref, w_ref = refs[_N_SPLIT:]
    wt = wt_ref[...]
    iota = jax.lax.broadcasted_iota(
        jnp.int32, (_STRIP, _N_EXPERTS), 1).astype(jnp.float32)
    for c in range(_N_SPLIT):
        x = x_refs[c][...]
        logits = jnp.dot(x, wt, preferred_element_type=jnp.float32)
        for t in range(_BS // _STRIP):
            strip = logits[t * _STRIP:(t + 1) * _STRIP, :]
            cur = strip
            vals = []
            idxs = []
            for k in range(_TOP_K):
                mv = jnp.max(cur, axis=-1, keepdims=True)
                ik = jnp.min(jnp.where(cur == mv, iota, float(_N_EXPERTS)),
                             axis=-1, keepdims=True)
                vals.append(mv)
                idxs.append(ik)
                if k + 1 < _TOP_K:
                    cur = jnp.where(iota == ik, -jnp.inf, cur)
            m = vals[0]
            e = jnp.exp(strip - m)
            s = jnp.sum(e, axis=-1, keepdims=True)
            probs = e / s
            rows = pl.ds(c * _BS + t * _STRIP, _STRIP)
            probs_ref[rows, :] = probs
            top_lv = jnp.concatenate(vals, axis=-1)
            top_idx = jnp.concatenate(idxs, axis=-1).astype(jnp.int32)
            top_vals = jnp.exp(top_lv - m) / s
            top_vals = top_vals / (jnp.sum(top_vals, axis=-1, keepdims=True)
                                   + 1e-9)
            idx_ref[rows, :] = top_idx
            w_ref[rows, :] = top_vals


def kernel(x, W):
    n_tokens = x.shape[0]
    grid = (n_tokens // _BT,)
    wt = W.T
    out_shapes = (
        jax.ShapeDtypeStruct((n_tokens, _N_EXPERTS), jnp.float32),
        jax.ShapeDtypeStruct((n_tokens, _TOP_K), jnp.int32),
        jax.ShapeDtypeStruct((n_tokens, _TOP_K), jnp.float32),
    )
    probs, idx, w = pl.pallas_call(
        _router_body,
        grid=grid,
        in_specs=[
            pl.BlockSpec((_BS, _D_MODEL),
                         lambda i, c=c: (_N_SPLIT * i + c, 0))
            for c in range(_N_SPLIT)
        ] + [
            pl.BlockSpec((_D_MODEL, _N_EXPERTS), lambda i: (0, 0)),
        ],
        out_specs=(
            pl.BlockSpec((_BT, _N_EXPERTS), lambda i: (i, 0)),
            pl.BlockSpec((_BT, _TOP_K), lambda i: (i, 0)),
            pl.BlockSpec((_BT, _TOP_K), lambda i: (i, 0)),
        ),
        out_shape=out_shapes,
        compiler_params=pltpu.CompilerParams(
            dimension_semantics=("arbitrary",),
        ),
    )(*([x] * _N_SPLIT), wt)
    return (idx, w, probs)
